# baseline (speedup 1.0000x reference)
import functools

import jax
import jax.numpy as jnp
from jax import lax
from jax.experimental import pallas as pl
from jax.experimental.pallas import tpu as pltpu
from jax.experimental.pallas import tpu_sc as plsc

N = 10000
E = 320000
D = 128

NC = 2
NS = 16
CHUNK = 128
KSUM = 158
K0 = 112
K1 = KSUM - K0
EP = NS * KSUM * CHUNK

NP = 10240
BN = 1024
ROWS_PER_TILE = NP // NS


def _sc_agg_body(xr_hbm, ew_hbm, out_hbm,
                 idx_v, buf_v, agg_sh, sem_i0, sem_i1, sem_g0, sem_g1):
    cid = lax.axis_index("c")
    sid = lax.axis_index("s")
    sem_i = (sem_i0, sem_i1)
    sem_g = (sem_g0, sem_g1)
    kcnt = jnp.where(cid == 0, K0, K1)
    base = jnp.where(cid == 0, sid * K0, NS * K0 + sid * K1)

    pltpu.async_copy(ew_hbm.at[base], idx_v.at[0], sem_i[0])
    pltpu.async_copy(ew_hbm.at[base + 1], idx_v.at[1], sem_i[1])

    zv = jnp.zeros((16,), jnp.float32)

    @pl.loop(0, CHUNK)
    def _(r):
        for c in range(D // 16):
            buf_v[0, r, pl.ds(c * 16, 16)] = zv

    r0 = sid * ROWS_PER_TILE
    for k in range(ROWS_PER_TILE // CHUNK):
        pltpu.sync_copy(buf_v.at[0],
                        agg_sh.at[pl.ds(r0 + k * CHUNK, CHUNK)])

    pltpu.make_async_copy(ew_hbm.at[base], idx_v.at[0], sem_i[0]).wait()
    pltpu.async_copy(xr_hbm.at[idx_v.at[0, 0]], buf_v.at[0], sem_g[0])
    plsc.subcore_barrier()


    @pl.loop(0, KSUM, step=2)
    def _(j0):
        for b in range(2):
            j = j0 + b
            nb = 1 - b

            @pl.when(j < kcnt)
            def _():
                @pl.when(j + 1 < kcnt)
                def _():
                    pltpu.make_async_copy(ew_hbm.at[base], idx_v.at[nb],
                                          sem_i[nb]).wait()
                    pltpu.async_copy(xr_hbm.at[idx_v.at[nb, 0]], buf_v.at[nb],
                                     sem_g[nb])

                pltpu.make_async_copy(xr_hbm.at[idx_v.at[b, 0]], buf_v.at[b],
                                      sem_g[b]).wait()
                pltpu.sync_copy(buf_v.at[b], agg_sh.at[idx_v.at[b, 1]],
                                add=True)

                @pl.when(j + 2 < kcnt)
                def _():
                    pltpu.async_copy(ew_hbm.at[base + j + 2], idx_v.at[b],
                                     sem_i[b])

    plsc.subcore_barrier()
    pltpu.sync_copy(agg_sh.at[pl.ds(r0, ROWS_PER_TILE)],
                    out_hbm.at[cid, pl.ds(r0, ROWS_PER_TILE)])


_sc_agg = pl.kernel(
    _sc_agg_body,
    out_type=jax.ShapeDtypeStruct((NC, NP, D), jnp.float32),
    mesh=plsc.VectorSubcoreMesh(core_axis_name="c", subcore_axis_name="s"),
    scratch_types=[
        pltpu.VMEM((2, 2, CHUNK), jnp.int32),
        pltpu.VMEM((2, CHUNK, D), jnp.float32),
        pltpu.VMEM_SHARED((NP, D), jnp.float32),
        pltpu.SemaphoreType.DMA,
        pltpu.SemaphoreType.DMA,
        pltpu.SemaphoreType.DMA,
        pltpu.SemaphoreType.DMA,
    ],
)


_PREC = jax.lax.Precision.HIGHEST


def _tc_first_body(x_ref, w_ref, o_ref):
    o_ref[...] = jnp.dot(x_ref[...], w_ref[...],
                         preferred_element_type=jnp.float32, precision=_PREC)


def _tc_mid_body(parts_ref, h_ref, wroot_ref, b_ref, wrel_ref,
                 hout_ref, xrout_ref, *, relu):
    acc = parts_ref[0] + parts_ref[1] + b_ref[...]
    acc += jnp.dot(h_ref[...], wroot_ref[...],
                   preferred_element_type=jnp.float32, precision=_PREC)
    if relu:
        acc = jnp.maximum(acc, 0.0)
    hout_ref[...] = acc
    xrout_ref[...] = jnp.dot(acc, wrel_ref[...],
                             preferred_element_type=jnp.float32,
                             precision=_PREC)


def _tc_last_body(parts_ref, h_ref, wroot_ref, b_ref, o_ref):
    acc = parts_ref[0] + parts_ref[1] + b_ref[...]
    acc += jnp.dot(h_ref[...], wroot_ref[...],
                   preferred_element_type=jnp.float32, precision=_PREC)
    o_ref[...] = acc


_row_spec = pl.BlockSpec((BN, D), lambda i: (i, 0))
_parts_spec = pl.BlockSpec((NC, BN, D), lambda i: (0, i, 0))
_w_spec = pl.BlockSpec((D, D), lambda i: (0, 0))
_b_spec = pl.BlockSpec((1, D), lambda i: (0, 0))
_GRID = (NP // BN,)

_tc_first = pl.pallas_call(
    _tc_first_body,
    grid=_GRID,
    in_specs=[_row_spec, _w_spec],
    out_specs=_row_spec,
    out_shape=jax.ShapeDtypeStruct((NP, D), jnp.float32),
)

_tc_mid_relu = pl.pallas_call(
    functools.partial(_tc_mid_body, relu=True),
    grid=_GRID,
    in_specs=[_parts_spec, _row_spec, _w_spec, _b_spec, _w_spec],
    out_specs=[_row_spec, _row_spec],
    out_shape=[jax.ShapeDtypeStruct((NP, D), jnp.float32),
               jax.ShapeDtypeStruct((NP, D), jnp.float32)],
)

_tc_last = pl.pallas_call(
    _tc_last_body,
    grid=_GRID,
    in_specs=[_parts_spec, _row_spec, _w_spec, _b_spec],
    out_specs=_row_spec,
    out_shape=jax.ShapeDtypeStruct((NP, D), jnp.float32),
)


def kernel(x, edge_index, W1_rel, W1_root, b1, W2_rel, W2_root, b2,
           W3_rel, W3_root, b3):
    pad = EP - E
    srcw = jnp.concatenate([edge_index[0], jnp.zeros((pad,), jnp.int32)])
    srcw = srcw.reshape(NS * KSUM, CHUNK)
    dstw = jnp.concatenate([edge_index[1], jnp.full((pad,), N, jnp.int32)])
    dstw = dstw.reshape(NS * KSUM, CHUNK)
    ew = jnp.stack([srcw, dstw], axis=1)
    xp = jnp.pad(x, ((0, NP - N), (0, 0)))
    b1r = b1.reshape(1, D)
    b2r = b2.reshape(1, D)
    b3r = b3.reshape(1, D)

    xr = _tc_first(xp, W1_rel)
    parts = _sc_agg(xr, ew)
    h, xr = _tc_mid_relu(parts, xp, W1_root, b1r, W2_rel)
    parts = _sc_agg(xr, ew)
    h, xr = _tc_mid_relu(parts, h, W2_root, b2r, W3_rel)
    parts = _sc_agg(xr, ew)
    out = _tc_last(parts, h, W3_root, b3r)
    return out[:N]


# speedup vs baseline: 1.0141x; 1.0141x over previous
import functools

import jax
import jax.numpy as jnp
from jax import lax
from jax.experimental import pallas as pl
from jax.experimental.pallas import tpu as pltpu
from jax.experimental.pallas import tpu_sc as plsc

N = 10000
E = 320000
D = 128

NC = 2
NS = 16
CHUNK = 128
KSUM = 158
K0 = 112
K1 = KSUM - K0
EP = NS * KSUM * CHUNK

NP = 10240
BN = 2048
ROWS_PER_TILE = NP // NS


def _sc_agg_body(xr_hbm, ew_hbm, out_hbm,
                 idx_v, buf_v, agg_sh, sem_i0, sem_i1, sem_g0, sem_g1):
    cid = lax.axis_index("c")
    sid = lax.axis_index("s")
    sem_i = (sem_i0, sem_i1)
    sem_g = (sem_g0, sem_g1)
    kcnt = jnp.where(cid == 0, K0, K1)
    base = jnp.where(cid == 0, sid * K0, NS * K0 + sid * K1)

    pltpu.async_copy(ew_hbm.at[base], idx_v.at[0], sem_i[0])
    pltpu.async_copy(ew_hbm.at[base + 1], idx_v.at[1], sem_i[1])

    zv = jnp.zeros((16,), jnp.float32)

    @pl.loop(0, CHUNK)
    def _(r):
        for c in range(D // 16):
            buf_v[0, r, pl.ds(c * 16, 16)] = zv

    r0 = sid * ROWS_PER_TILE
    for k in range(ROWS_PER_TILE // CHUNK):
        pltpu.sync_copy(buf_v.at[0],
                        agg_sh.at[pl.ds(r0 + k * CHUNK, CHUNK)])

    pltpu.make_async_copy(ew_hbm.at[base], idx_v.at[0], sem_i[0]).wait()
    pltpu.async_copy(xr_hbm.at[idx_v.at[0, 0]], buf_v.at[0], sem_g[0])
    plsc.subcore_barrier()


    @pl.loop(0, KSUM, step=2)
    def _(j0):
        for b in range(2):
            j = j0 + b
            nb = 1 - b

            @pl.when(j < kcnt)
            def _():
                @pl.when(j + 1 < kcnt)
                def _():
                    pltpu.make_async_copy(ew_hbm.at[base], idx_v.at[nb],
                                          sem_i[nb]).wait()
                    pltpu.async_copy(xr_hbm.at[idx_v.at[nb, 0]], buf_v.at[nb],
                                     sem_g[nb])

                pltpu.make_async_copy(xr_hbm.at[idx_v.at[b, 0]], buf_v.at[b],
                                      sem_g[b]).wait()
                pltpu.sync_copy(buf_v.at[b], agg_sh.at[idx_v.at[b, 1]],
                                add=True)

                @pl.when(j + 2 < kcnt)
                def _():
                    pltpu.async_copy(ew_hbm.at[base + j + 2], idx_v.at[b],
                                     sem_i[b])

    plsc.subcore_barrier()
    pltpu.sync_copy(agg_sh.at[pl.ds(r0, ROWS_PER_TILE)],
                    out_hbm.at[cid, pl.ds(r0, ROWS_PER_TILE)])


_sc_agg = pl.kernel(
    _sc_agg_body,
    out_type=jax.ShapeDtypeStruct((NC, NP, D), jnp.float32),
    mesh=plsc.VectorSubcoreMesh(core_axis_name="c", subcore_axis_name="s"),
    scratch_types=[
        pltpu.VMEM((2, 2, CHUNK), jnp.int32),
        pltpu.VMEM((2, CHUNK, D), jnp.float32),
        pltpu.VMEM_SHARED((NP, D), jnp.float32),
        pltpu.SemaphoreType.DMA,
        pltpu.SemaphoreType.DMA,
        pltpu.SemaphoreType.DMA,
        pltpu.SemaphoreType.DMA,
    ],
)


_PREC = jax.lax.Precision.HIGHEST


def _tc_first_body(x_ref, w_ref, o_ref):
    o_ref[...] = jnp.dot(x_ref[...], w_ref[...],
                         preferred_element_type=jnp.float32, precision=_PREC)


def _tc_mid_body(parts_ref, h_ref, wroot_ref, b_ref, wrel_ref,
                 hout_ref, xrout_ref, *, relu):
    acc = parts_ref[0] + parts_ref[1] + b_ref[...]
    acc += jnp.dot(h_ref[...], wroot_ref[...],
                   preferred_element_type=jnp.float32, precision=_PREC)
    if relu:
        acc = jnp.maximum(acc, 0.0)
    hout_ref[...] = acc
    xrout_ref[...] = jnp.dot(acc, wrel_ref[...],
                             preferred_element_type=jnp.float32,
                             precision=_PREC)


def _tc_last_body(parts_ref, h_ref, wroot_ref, b_ref, o_ref):
    acc = parts_ref[0] + parts_ref[1] + b_ref[...]
    acc += jnp.dot(h_ref[...], wroot_ref[...],
                   preferred_element_type=jnp.float32, precision=_PREC)
    o_ref[...] = acc


_row_spec = pl.BlockSpec((BN, D), lambda i: (i, 0))
_parts_spec = pl.BlockSpec((NC, BN, D), lambda i: (0, i, 0))
_w_spec = pl.BlockSpec((D, D), lambda i: (0, 0))
_b_spec = pl.BlockSpec((1, D), lambda i: (0, 0))
_GRID = (NP // BN,)

_tc_first = pl.pallas_call(
    _tc_first_body,
    grid=_GRID,
    in_specs=[_row_spec, _w_spec],
    out_specs=_row_spec,
    out_shape=jax.ShapeDtypeStruct((NP, D), jnp.float32),
)

_tc_mid_relu = pl.pallas_call(
    functools.partial(_tc_mid_body, relu=True),
    grid=_GRID,
    in_specs=[_parts_spec, _row_spec, _w_spec, _b_spec, _w_spec],
    out_specs=[_row_spec, _row_spec],
    out_shape=[jax.ShapeDtypeStruct((NP, D), jnp.float32),
               jax.ShapeDtypeStruct((NP, D), jnp.float32)],
)

_BL = 2000
_tc_last = pl.pallas_call(
    _tc_last_body,
    grid=(N // _BL,),
    in_specs=[pl.BlockSpec((NC, _BL, D), lambda i: (0, i, 0)),
              pl.BlockSpec((_BL, D), lambda i: (i, 0)),
              _w_spec, _b_spec],
    out_specs=pl.BlockSpec((_BL, D), lambda i: (i, 0)),
    out_shape=jax.ShapeDtypeStruct((N, D), jnp.float32),
)


def kernel(x, edge_index, W1_rel, W1_root, b1, W2_rel, W2_root, b2,
           W3_rel, W3_root, b3):
    pad = EP - E
    srcw = jnp.concatenate([edge_index[0], jnp.zeros((pad,), jnp.int32)])
    srcw = srcw.reshape(NS * KSUM, CHUNK)
    dstw = jnp.concatenate([edge_index[1], jnp.full((pad,), N, jnp.int32)])
    dstw = dstw.reshape(NS * KSUM, CHUNK)
    ew = jnp.stack([srcw, dstw], axis=1)
    xp = jnp.pad(x, ((0, NP - N), (0, 0)))
    b1r = b1.reshape(1, D)
    b2r = b2.reshape(1, D)
    b3r = b3.reshape(1, D)

    xr = _tc_first(xp, W1_rel)
    parts = _sc_agg(xr, ew)
    h, xr = _tc_mid_relu(parts, xp, W1_root, b1r, W2_rel)
    parts = _sc_agg(xr, ew)
    h, xr = _tc_mid_relu(parts, h, W2_root, b2r, W3_rel)
    parts = _sc_agg(xr, ew)
    return _tc_last(parts, h, W3_root, b3r)
